# baseline (speedup 1.0000x reference)
import functools

import jax
import jax.numpy as jnp
from jax import lax
from jax.experimental import pallas as pl
from jax.experimental.pallas import tpu as pltpu
from jax.experimental.pallas import tpu_sc as plsc

_E = 8
_D_IN = 768
_D_OUT = 768
_D_FF = 4 * _D_IN
_N = 2048
_CAP = _N
_B = 1024
_TPE = _CAP // _B
_MAX_S = 11


def _router_body(x_ref, gw_ref, gb_ref,
                 destA_ref, destB_ref, wA_ref, wB_ref, se_ref, st_ref,
                 act_ref):
    n = x_ref.shape[0]
    xv = x_ref[...]
    logits = lax.dot_general(
        xv.astype(jnp.bfloat16), gw_ref[...].astype(jnp.bfloat16),
        (((1,), (1,)), ((), ())),
        preferred_element_type=jnp.float32) + gb_ref[...]
    col = lax.broadcasted_iota(jnp.int32, (n, _E), 1)
    m1 = jnp.max(logits, axis=1, keepdims=True)
    i1 = jnp.min(jnp.where(logits == m1, col, _E), axis=1, keepdims=True)
    mask1 = col == i1
    logits2 = jnp.where(mask1, -1e30, logits)
    m2 = jnp.max(logits2, axis=1, keepdims=True)
    i2 = jnp.min(jnp.where(logits2 == m2, col, _E), axis=1, keepdims=True)
    mask2 = col == i2
    e21 = jnp.exp(m2 - m1)
    w_hi = 1.0 / (1.0 + e21)
    wA_ref[...] = w_hi
    wB_ref[...] = 1.0 - w_hi

    onehot = (mask1 | mask2).astype(jnp.bfloat16)
    rowi = lax.broadcasted_iota(jnp.int32, (n, n), 0)
    coli = lax.broadcasted_iota(jnp.int32, (n, n), 1)
    tril = (coli < rowi).astype(jnp.bfloat16)
    pos = lax.dot_general(tril, onehot, (((1,), (0,)), ((), ())),
                          preferred_element_type=jnp.float32)
    posA = jnp.sum(jnp.where(mask1, pos, 0.0), axis=1, keepdims=True)
    posB = jnp.sum(jnp.where(mask2, pos, 0.0), axis=1, keepdims=True)
    destA_ref[...] = i1 * _CAP + posA.astype(jnp.int32)
    destB_ref[...] = i2 * _CAP + posB.astype(jnp.int32)

    counts = jnp.sum(onehot.astype(jnp.float32), axis=0, keepdims=True)
    tiles = (counts.astype(jnp.int32) + (_B - 1)) // _B
    tri8l = (lax.broadcasted_iota(jnp.int32, (_E, _E), 1)
             < lax.broadcasted_iota(jnp.int32, (_E, _E), 0)).astype(jnp.float32)
    cum_ex = lax.dot_general(tiles.astype(jnp.float32), tri8l,
                             (((1,), (0,)), ((), ())),
                             preferred_element_type=jnp.float32
                             ).astype(jnp.int32)
    cum_in = cum_ex + tiles
    total = jnp.sum(tiles)
    s_eff = jnp.minimum(lax.broadcasted_iota(jnp.int32, (_MAX_S, 1), 0),
                        total - 1)
    ind = ((cum_ex <= s_eff) & (s_eff < cum_in)).astype(jnp.int32)
    eio = lax.broadcasted_iota(jnp.int32, (_MAX_S, _E), 1)
    se = jnp.sum(ind * eio, axis=1, keepdims=True)
    st = s_eff - jnp.sum(ind * cum_ex, axis=1, keepdims=True)
    se_ref[...] = se
    st_ref[...] = st
    act_ref[...] = (lax.broadcasted_iota(jnp.int32, (_MAX_S, 1), 0)
                    < total).astype(jnp.int32)


def _router(x, gate_w, gate_b):
    n = x.shape[0]
    return pl.pallas_call(
        _router_body,
        in_specs=[
            pl.BlockSpec((n, _D_IN), lambda: (0, 0)),
            pl.BlockSpec((_E, _D_IN), lambda: (0, 0)),
            pl.BlockSpec((1, _E), lambda: (0, 0)),
        ],
        out_specs=[
            pl.BlockSpec((n, 1), lambda: (0, 0)),
            pl.BlockSpec((n, 1), lambda: (0, 0)),
            pl.BlockSpec((n, 1), lambda: (0, 0)),
            pl.BlockSpec((n, 1), lambda: (0, 0)),
            pl.BlockSpec((_MAX_S, 1), lambda: (0, 0)),
            pl.BlockSpec((_MAX_S, 1), lambda: (0, 0)),
            pl.BlockSpec((_MAX_S, 1), lambda: (0, 0)),
        ],
        out_shape=[
            jax.ShapeDtypeStruct((n, 1), jnp.int32),
            jax.ShapeDtypeStruct((n, 1), jnp.int32),
            jax.ShapeDtypeStruct((n, 1), jnp.float32),
            jax.ShapeDtypeStruct((n, 1), jnp.float32),
            jax.ShapeDtypeStruct((_MAX_S, 1), jnp.int32),
            jax.ShapeDtypeStruct((_MAX_S, 1), jnp.int32),
            jax.ShapeDtypeStruct((_MAX_S, 1), jnp.int32),
        ],
    )(x, gate_w, gate_b.reshape(1, _E))


def _dispatch(x, destA, destB):
    info = plsc.get_sparse_core_info()
    nw = info.num_cores * info.num_subcores
    per = _N // nw

    @functools.partial(
        pl.kernel,
        out_type=jax.ShapeDtypeStruct((_E * _CAP, _D_IN), jnp.float32),
        mesh=plsc.VectorSubcoreMesh(core_axis_name="c", subcore_axis_name="s"),
        scratch_types=[
            pltpu.VMEM((per,), jnp.int32),
            pltpu.VMEM((per,), jnp.int32),
            pltpu.VMEM((per, _D_IN), jnp.float32),
            pltpu.SemaphoreType.DMA,
        ],
    )
    def k2(x_hbm, dA_hbm, dB_hbm, xg_hbm, idxA_v, idxB_v, rows_v, semr):
        wid = lax.axis_index("s") * info.num_cores + lax.axis_index("c")
        base = wid * per
        cpA = pltpu.async_copy(dA_hbm.at[pl.ds(base, per)], idxA_v, semr)
        cpB = pltpu.async_copy(dB_hbm.at[pl.ds(base, per)], idxB_v, semr)
        pltpu.sync_copy(x_hbm.at[pl.ds(base, per)], rows_v)
        cpA.wait()
        cpB.wait()
        pltpu.async_copy(rows_v, xg_hbm.at[idxA_v], semr).wait()
        pltpu.async_copy(rows_v, xg_hbm.at[idxB_v], semr).wait()

    return k2(x, destA, destB)


def _gmm_body(se_ref, st_ref, act_ref, xg_ref, w1_ref, w2_ref, b1_ref, b2_ref,
              y_ref):
    s = pl.program_id(0)
    e = se_ref[s]

    @pl.when(act_ref[s] == 1)
    def _step():
        xb = xg_ref[...].astype(jnp.bfloat16)
        h = lax.dot_general(xb, w1_ref[0].astype(jnp.bfloat16),
                            (((1,), (1,)), ((), ())),
                            preferred_element_type=jnp.float32)
        h = jnp.maximum(h + b1_ref[pl.ds(e, 1), :], 0.0)
        y = lax.dot_general(h.astype(jnp.bfloat16),
                            w2_ref[0].astype(jnp.bfloat16),
                            (((1,), (1,)), ((), ())),
                            preferred_element_type=jnp.float32)
        y_ref[...] = y + b2_ref[pl.ds(e, 1), :]


def _gmm(se, st, act, xg, w1, w2, b1, b2):
    grid_spec = pltpu.PrefetchScalarGridSpec(
        num_scalar_prefetch=3,
        grid=(_MAX_S,),
        in_specs=[
            pl.BlockSpec((_B, _D_IN),
                         lambda s, se, st, act: (se[s] * _TPE + st[s], 0)),
            pl.BlockSpec((1, _D_FF, _D_IN),
                         lambda s, se, st, act: (se[s], 0, 0)),
            pl.BlockSpec((1, _D_OUT, _D_FF),
                         lambda s, se, st, act: (se[s], 0, 0)),
            pl.BlockSpec((_E, _D_FF), lambda s, se, st, act: (0, 0)),
            pl.BlockSpec((_E, _D_OUT), lambda s, se, st, act: (0, 0)),
        ],
        out_specs=pl.BlockSpec((_B, _D_OUT),
                               lambda s, se, st, act: (se[s] * _TPE + st[s], 0)),
        scratch_shapes=[],
    )
    return pl.pallas_call(
        _gmm_body,
        grid_spec=grid_spec,
        out_shape=jax.ShapeDtypeStruct((_E * _CAP, _D_OUT), jnp.float32),
        compiler_params=pltpu.CompilerParams(
            dimension_semantics=("arbitrary",)),
    )(se, st, act, xg, w1, w2, b1, b2)


def _collect_combine(ylist, destA, destB, wAf, wBf):
    info = plsc.get_sparse_core_info()
    nw = info.num_cores * info.num_subcores
    per = _N // nw
    nlane = info.num_lanes

    @functools.partial(
        pl.kernel,
        out_type=jax.ShapeDtypeStruct((_N, _D_OUT), jnp.float32),
        mesh=plsc.VectorSubcoreMesh(core_axis_name="c", subcore_axis_name="s"),
        scratch_types=[
            pltpu.VMEM((per,), jnp.int32),
            pltpu.VMEM((per,), jnp.int32),
            pltpu.VMEM((per + 16,), jnp.float32),
            pltpu.VMEM((per + 16,), jnp.float32),
            pltpu.VMEM((per, _D_OUT), jnp.float32),
            pltpu.VMEM((per, _D_OUT), jnp.float32),
            pltpu.SemaphoreType.DMA,
        ],
    )
    def k4(yl_hbm, dA_hbm, dB_hbm, wA_hbm, wB_hbm, out_hbm,
           idxA_v, idxB_v, wA_v, wB_v, rowsA_v, rowsB_v, semr):
        wid = lax.axis_index("s") * info.num_cores + lax.axis_index("c")
        base = wid * per
        cps = [pltpu.async_copy(dA_hbm.at[pl.ds(base, per)], idxA_v, semr),
               pltpu.async_copy(dB_hbm.at[pl.ds(base, per)], idxB_v, semr),
               pltpu.async_copy(wA_hbm.at[pl.ds(base, per)],
                                wA_v.at[pl.ds(0, per)], semr),
               pltpu.async_copy(wB_hbm.at[pl.ds(base, per)],
                                wB_v.at[pl.ds(0, per)], semr)]
        for cp in cps:
            cp.wait()
        gA = pltpu.async_copy(yl_hbm.at[idxA_v], rowsA_v, semr)
        gB = pltpu.async_copy(yl_hbm.at[idxB_v], rowsB_v, semr)
        gA.wait()
        gB.wait()

        def tok(r, carry):
            wa = wA_v[pl.ds(r, nlane)][0]
            wb = wB_v[pl.ds(r, nlane)][0]
            for k in range(_D_OUT // nlane):
                a = rowsA_v[r, pl.ds(k * nlane, nlane)]
                b = rowsB_v[r, pl.ds(k * nlane, nlane)]
                rowsA_v[r, pl.ds(k * nlane, nlane)] = wa * a + wb * b
            return carry

        lax.fori_loop(0, per, tok, 0)
        pltpu.sync_copy(rowsA_v, out_hbm.at[pl.ds(base, per)])

    return k4(ylist, destA, destB, wAf, wBf)


def kernel(x, gate_w, gate_b, w1, b1, w2, b2):
    destA2, destB2, wA, wB, se2, st2, act2 = _router(x, gate_w, gate_b)
    destA = destA2.reshape(_N)
    destB = destB2.reshape(_N)
    se = se2.reshape(_MAX_S)
    st = st2.reshape(_MAX_S)
    act = act2.reshape(_MAX_S)
    xg = _dispatch(x, destA, destB)
    ylist = _gmm(se, st, act, xg, w1, w2, b1, b2)
    return _collect_combine(ylist, destA, destB,
                            wA.reshape(_N), wB.reshape(_N))


# speedup vs baseline: 1.0603x; 1.0603x over previous
import functools

import jax
import jax.numpy as jnp
from jax import lax
from jax.experimental import pallas as pl
from jax.experimental.pallas import tpu as pltpu
from jax.experimental.pallas import tpu_sc as plsc

_E = 8
_D_IN = 768
_D_OUT = 768
_D_FF = 4 * _D_IN
_N = 2048
_CAP = _N
_B = 512
_TPE = _CAP // _B
_MAX_S = 15


def _router_body(x_ref, gw_ref, gb_ref,
                 destA_ref, destB_ref, wA_ref, wB_ref, se_ref, st_ref,
                 act_ref):
    n = x_ref.shape[0]
    xv = x_ref[...]
    logits = lax.dot_general(
        xv.astype(jnp.bfloat16), gw_ref[...].astype(jnp.bfloat16),
        (((1,), (1,)), ((), ())),
        preferred_element_type=jnp.float32) + gb_ref[...]
    col = lax.broadcasted_iota(jnp.int32, (n, _E), 1)
    m1 = jnp.max(logits, axis=1, keepdims=True)
    i1 = jnp.min(jnp.where(logits == m1, col, _E), axis=1, keepdims=True)
    mask1 = col == i1
    logits2 = jnp.where(mask1, -1e30, logits)
    m2 = jnp.max(logits2, axis=1, keepdims=True)
    i2 = jnp.min(jnp.where(logits2 == m2, col, _E), axis=1, keepdims=True)
    mask2 = col == i2
    e21 = jnp.exp(m2 - m1)
    w_hi = 1.0 / (1.0 + e21)
    wA_ref[...] = w_hi
    wB_ref[...] = 1.0 - w_hi

    onehot = (mask1 | mask2).astype(jnp.bfloat16)
    rowi = lax.broadcasted_iota(jnp.int32, (n, n), 0)
    coli = lax.broadcasted_iota(jnp.int32, (n, n), 1)
    tril = (coli < rowi).astype(jnp.bfloat16)
    pos = lax.dot_general(tril, onehot, (((1,), (0,)), ((), ())),
                          preferred_element_type=jnp.float32)
    posA = jnp.sum(jnp.where(mask1, pos, 0.0), axis=1, keepdims=True)
    posB = jnp.sum(jnp.where(mask2, pos, 0.0), axis=1, keepdims=True)
    destA_ref[...] = i1 * _CAP + posA.astype(jnp.int32)
    destB_ref[...] = i2 * _CAP + posB.astype(jnp.int32)

    counts = jnp.sum(onehot.astype(jnp.float32), axis=0, keepdims=True)
    tiles = (counts.astype(jnp.int32) + (_B - 1)) // _B
    tri8l = (lax.broadcasted_iota(jnp.int32, (_E, _E), 1)
             < lax.broadcasted_iota(jnp.int32, (_E, _E), 0)).astype(jnp.float32)
    cum_ex = lax.dot_general(tiles.astype(jnp.float32), tri8l,
                             (((1,), (0,)), ((), ())),
                             preferred_element_type=jnp.float32
                             ).astype(jnp.int32)
    cum_in = cum_ex + tiles
    total = jnp.sum(tiles)
    s_eff = jnp.minimum(lax.broadcasted_iota(jnp.int32, (_MAX_S, 1), 0),
                        total - 1)
    ind = ((cum_ex <= s_eff) & (s_eff < cum_in)).astype(jnp.int32)
    eio = lax.broadcasted_iota(jnp.int32, (_MAX_S, _E), 1)
    se = jnp.sum(ind * eio, axis=1, keepdims=True)
    st = s_eff - jnp.sum(ind * cum_ex, axis=1, keepdims=True)
    se_ref[...] = se
    st_ref[...] = st
    act_ref[...] = (lax.broadcasted_iota(jnp.int32, (_MAX_S, 1), 0)
                    < total).astype(jnp.int32)


def _router(x, gate_w, gate_b):
    n = x.shape[0]
    return pl.pallas_call(
        _router_body,
        in_specs=[
            pl.BlockSpec((n, _D_IN), lambda: (0, 0)),
            pl.BlockSpec((_E, _D_IN), lambda: (0, 0)),
            pl.BlockSpec((1, _E), lambda: (0, 0)),
        ],
        out_specs=[
            pl.BlockSpec((n, 1), lambda: (0, 0)),
            pl.BlockSpec((n, 1), lambda: (0, 0)),
            pl.BlockSpec((n, 1), lambda: (0, 0)),
            pl.BlockSpec((n, 1), lambda: (0, 0)),
            pl.BlockSpec((_MAX_S, 1), lambda: (0, 0)),
            pl.BlockSpec((_MAX_S, 1), lambda: (0, 0)),
            pl.BlockSpec((_MAX_S, 1), lambda: (0, 0)),
        ],
        out_shape=[
            jax.ShapeDtypeStruct((n, 1), jnp.int32),
            jax.ShapeDtypeStruct((n, 1), jnp.int32),
            jax.ShapeDtypeStruct((n, 1), jnp.float32),
            jax.ShapeDtypeStruct((n, 1), jnp.float32),
            jax.ShapeDtypeStruct((_MAX_S, 1), jnp.int32),
            jax.ShapeDtypeStruct((_MAX_S, 1), jnp.int32),
            jax.ShapeDtypeStruct((_MAX_S, 1), jnp.int32),
        ],
    )(x, gate_w, gate_b.reshape(1, _E))


def _dispatch(x, destA, destB):
    info = plsc.get_sparse_core_info()
    nw = info.num_cores * info.num_subcores
    per = _N // nw

    @functools.partial(
        pl.kernel,
        out_type=jax.ShapeDtypeStruct((_E * _CAP, _D_IN), jnp.float32),
        mesh=plsc.VectorSubcoreMesh(core_axis_name="c", subcore_axis_name="s"),
        scratch_types=[
            pltpu.VMEM((per,), jnp.int32),
            pltpu.VMEM((per,), jnp.int32),
            pltpu.VMEM((per, _D_IN), jnp.float32),
            pltpu.SemaphoreType.DMA,
        ],
    )
    def k2(x_hbm, dA_hbm, dB_hbm, xg_hbm, idxA_v, idxB_v, rows_v, semr):
        wid = lax.axis_index("s") * info.num_cores + lax.axis_index("c")
        base = wid * per
        cpA = pltpu.async_copy(dA_hbm.at[pl.ds(base, per)], idxA_v, semr)
        cpB = pltpu.async_copy(dB_hbm.at[pl.ds(base, per)], idxB_v, semr)
        pltpu.sync_copy(x_hbm.at[pl.ds(base, per)], rows_v)
        cpA.wait()
        cpB.wait()
        pltpu.async_copy(rows_v, xg_hbm.at[idxA_v], semr).wait()
        pltpu.async_copy(rows_v, xg_hbm.at[idxB_v], semr).wait()

    return k2(x, destA, destB)


def _gmm_body(se_ref, st_ref, act_ref, xg_ref, w1_ref, w2_ref, b1_ref, b2_ref,
              y_ref):
    s = pl.program_id(0)
    e = se_ref[s]

    @pl.when(act_ref[s] == 1)
    def _step():
        xb = xg_ref[...].astype(jnp.bfloat16)
        h = lax.dot_general(xb, w1_ref[0].astype(jnp.bfloat16),
                            (((1,), (1,)), ((), ())),
                            preferred_element_type=jnp.float32)
        h = jnp.maximum(h + b1_ref[pl.ds(e, 1), :], 0.0)
        y = lax.dot_general(h.astype(jnp.bfloat16),
                            w2_ref[0].astype(jnp.bfloat16),
                            (((1,), (1,)), ((), ())),
                            preferred_element_type=jnp.float32)
        y_ref[...] = y + b2_ref[pl.ds(e, 1), :]


def _gmm(se, st, act, xg, w1, w2, b1, b2):
    grid_spec = pltpu.PrefetchScalarGridSpec(
        num_scalar_prefetch=3,
        grid=(_MAX_S,),
        in_specs=[
            pl.BlockSpec((_B, _D_IN),
                         lambda s, se, st, act: (se[s] * _TPE + st[s], 0)),
            pl.BlockSpec((1, _D_FF, _D_IN),
                         lambda s, se, st, act: (se[s], 0, 0)),
            pl.BlockSpec((1, _D_OUT, _D_FF),
                         lambda s, se, st, act: (se[s], 0, 0)),
            pl.BlockSpec((_E, _D_FF), lambda s, se, st, act: (0, 0)),
            pl.BlockSpec((_E, _D_OUT), lambda s, se, st, act: (0, 0)),
        ],
        out_specs=pl.BlockSpec((_B, _D_OUT),
                               lambda s, se, st, act: (se[s] * _TPE + st[s], 0)),
        scratch_shapes=[],
    )
    return pl.pallas_call(
        _gmm_body,
        grid_spec=grid_spec,
        out_shape=jax.ShapeDtypeStruct((_E * _CAP, _D_OUT), jnp.float32),
        compiler_params=pltpu.CompilerParams(
            dimension_semantics=("arbitrary",)),
    )(se, st, act, xg, w1, w2, b1, b2)


def _collect_combine(ylist, destA, destB, wAf, wBf):
    info = plsc.get_sparse_core_info()
    nw = info.num_cores * info.num_subcores
    per = _N // nw
    nlane = info.num_lanes

    @functools.partial(
        pl.kernel,
        out_type=jax.ShapeDtypeStruct((_N, _D_OUT), jnp.float32),
        mesh=plsc.VectorSubcoreMesh(core_axis_name="c", subcore_axis_name="s"),
        scratch_types=[
            pltpu.VMEM((per,), jnp.int32),
            pltpu.VMEM((per,), jnp.int32),
            pltpu.VMEM((per + 16,), jnp.float32),
            pltpu.VMEM((per + 16,), jnp.float32),
            pltpu.VMEM((per, _D_OUT), jnp.float32),
            pltpu.VMEM((per, _D_OUT), jnp.float32),
            pltpu.SemaphoreType.DMA,
        ],
    )
    def k4(yl_hbm, dA_hbm, dB_hbm, wA_hbm, wB_hbm, out_hbm,
           idxA_v, idxB_v, wA_v, wB_v, rowsA_v, rowsB_v, semr):
        wid = lax.axis_index("s") * info.num_cores + lax.axis_index("c")
        base = wid * per
        cps = [pltpu.async_copy(dA_hbm.at[pl.ds(base, per)], idxA_v, semr),
               pltpu.async_copy(dB_hbm.at[pl.ds(base, per)], idxB_v, semr),
               pltpu.async_copy(wA_hbm.at[pl.ds(base, per)],
                                wA_v.at[pl.ds(0, per)], semr),
               pltpu.async_copy(wB_hbm.at[pl.ds(base, per)],
                                wB_v.at[pl.ds(0, per)], semr)]
        for cp in cps:
            cp.wait()
        gA = pltpu.async_copy(yl_hbm.at[idxA_v], rowsA_v, semr)
        gB = pltpu.async_copy(yl_hbm.at[idxB_v], rowsB_v, semr)
        gA.wait()
        gB.wait()

        def tok(r, carry):
            wa = wA_v[pl.ds(r, nlane)][0]
            wb = wB_v[pl.ds(r, nlane)][0]
            for k in range(_D_OUT // nlane):
                a = rowsA_v[r, pl.ds(k * nlane, nlane)]
                b = rowsB_v[r, pl.ds(k * nlane, nlane)]
                rowsA_v[r, pl.ds(k * nlane, nlane)] = wa * a + wb * b
            return carry

        lax.fori_loop(0, per, tok, 0)
        pltpu.sync_copy(rowsA_v, out_hbm.at[pl.ds(base, per)])

    return k4(ylist, destA, destB, wAf, wBf)


def kernel(x, gate_w, gate_b, w1, b1, w2, b2):
    destA2, destB2, wA, wB, se2, st2, act2 = _router(x, gate_w, gate_b)
    destA = destA2.reshape(_N)
    destB = destB2.reshape(_N)
    se = se2.reshape(_MAX_S)
    st = st2.reshape(_MAX_S)
    act = act2.reshape(_MAX_S)
    xg = _dispatch(x, destA, destB)
    ylist = _gmm(se, st, act, xg, w1, w2, b1, b2)
    return _collect_combine(ylist, destA, destB,
                            wA.reshape(_N), wB.reshape(_N))
